# baseline (speedup 1.0000x reference)
import jax
import jax.numpy as jnp
from jax import lax
from jax.experimental import pallas as pl
from jax.experimental.pallas import tpu as pltpu
from jax.experimental.pallas import tpu_sc as plsc

B = 1024
L_DOC = 200
L_Q = 50
LT = L_DOC + L_Q
D = 64
NW = 32
TPW = B * LT // NW
CH = 80
NCH = TPW // CH
NBUF = 5
PEXT = CH * ((LT + CH - 1) // CH) + CH


def _emb_kernel(ids_hbm, pos_hbm, table_hbm, out_hbm,
                idx_v, pos_v, rows_g, rows_o, *sems):
    gsems = sems[:NBUF]
    osems = sems[NBUF:]
    wid = lax.axis_index("s") * 2 + lax.axis_index("c")
    base = wid * TPW
    pltpu.sync_copy(ids_hbm.at[wid], idx_v)
    pltpu.sync_copy(pos_hbm, pos_v)

    def gather(c, b):
        return pltpu.make_async_copy(
            table_hbm.at[idx_v.at[c]], rows_g.at[b], gsems[b])

    def out_copy(c, b):
        return pltpu.make_async_copy(
            rows_o.at[b], out_hbm.at[pl.ds(base + c * CH, CH)], osems[b])

    for b in range(NBUF):
        gather(b, b).start()

    def outer(c0, carry):
        for b in range(NBUF):
            c = c0 * NBUF + b
            gather(c, b).wait()

            @pl.when(c0 > 0)
            def _wait_prev():
                out_copy(c - NBUF, b).wait()

            poff = lax.rem(c * CH, LT)

            def add_body(r, carry2):
                for j in range(4):
                    sl = pl.ds(j * 16, 16)
                    rows_o[b, r, sl] = rows_g[b, r, sl] + pos_v[poff + r, sl]
                return carry2

            lax.fori_loop(0, CH, add_body, 0)

            @pl.when(c + NBUF < NCH)
            def _next_gather():
                gather(c + NBUF, b).start()

            out_copy(c, b).start()
        return carry

    lax.fori_loop(0, NCH // NBUF, outer, 0)
    for b in range(NBUF):
        out_copy(NCH - NBUF + b, b).wait()


def kernel(x1_ids, x2_ids, emb_table, pos_table):
    ids = jnp.concatenate([x1_ids, x2_ids], axis=1).astype(jnp.int32)
    ids_r = ids.reshape(NW, NCH, CH)
    pos_full = jnp.concatenate([pos_table[:L_DOC], pos_table[:L_Q]], axis=0)
    pos_ext = jnp.concatenate([pos_full, pos_full[: PEXT - LT]], axis=0)
    out = pl.kernel(
        _emb_kernel,
        out_type=jax.ShapeDtypeStruct((B * LT, D), jnp.float32),
        mesh=plsc.VectorSubcoreMesh(core_axis_name="c", subcore_axis_name="s"),
        compiler_params=pltpu.CompilerParams(use_tc_tiling_on_sc=False),
        scratch_types=[
            pltpu.VMEM((NCH, CH), jnp.int32),
            pltpu.VMEM((PEXT, D), jnp.float32),
            pltpu.VMEM((NBUF, CH, D), jnp.float32),
            pltpu.VMEM((NBUF, CH, D), jnp.float32),
        ] + [pltpu.SemaphoreType.DMA] * (2 * NBUF),
    )(ids_r, pos_ext, emb_table)
    return out.reshape(B, LT, D)
